# Initial kernel scaffold; baseline (speedup 1.0000x reference)
import jax
import jax.numpy as jnp
from jax.experimental import pallas as pl


def kernel(questions_tensor, table):
    raise NotImplementedError("write your pallas kernel here")


# speedup vs baseline: 1.8436x; 1.8436x over previous
import functools

import jax
import jax.numpy as jnp
from jax import lax
from jax.experimental import pallas as pl
from jax.experimental.pallas import tpu as pltpu
from jax.experimental.pallas import tpu_sc as plsc

BATCH = 16384
SEQ = 50
EMBED_DIM = 64
TOTAL = BATCH * SEQ

_info = plsc.get_sparse_core_info()
_NC = _info.num_cores
_NS = _info.num_subcores
_NW = _NC * _NS
_BPW = TOTAL // _NW
_CHUNK = 1024
_NCHUNK = _BPW // _CHUNK

_mesh = plsc.VectorSubcoreMesh(core_axis_name="c", subcore_axis_name="s")


@functools.partial(
    pl.kernel,
    mesh=_mesh,
    out_type=jax.ShapeDtypeStruct((TOTAL, EMBED_DIM), jnp.float32),
    scratch_types=[
        pltpu.VMEM((_CHUNK,), jnp.int32),
        pltpu.VMEM((_CHUNK, EMBED_DIM), jnp.float32),
        pltpu.SemaphoreType.DMA,
    ],
    compiler_params=pltpu.CompilerParams(use_tc_tiling_on_sc=False),
)
def _emb_lookup(idx_hbm, table_hbm, out_hbm, idx_v, rows_v, sem):
    wid = lax.axis_index("s") * _NC + lax.axis_index("c")
    base0 = wid * _BPW

    def body(i, carry):
        base = base0 + i * _CHUNK
        pltpu.sync_copy(idx_hbm.at[pl.ds(base, _CHUNK)], idx_v)
        pltpu.async_copy(table_hbm.at[idx_v], rows_v, sem).wait()
        pltpu.sync_copy(rows_v, out_hbm.at[pl.ds(base, _CHUNK)])
        return carry

    lax.fori_loop(0, _NCHUNK, body, 0)


def kernel(questions_tensor, table):
    flat_idx = questions_tensor.reshape(TOTAL)
    out = _emb_lookup(flat_idx, table)
    return out.reshape(BATCH, SEQ, EMBED_DIM)


# speedup vs baseline: 1.8732x; 1.0161x over previous
import functools

import jax
import jax.numpy as jnp
from jax import lax
from jax.experimental import pallas as pl
from jax.experimental.pallas import tpu as pltpu
from jax.experimental.pallas import tpu_sc as plsc

BATCH = 16384
SEQ = 50
EMBED_DIM = 64
TOTAL = BATCH * SEQ

_info = plsc.get_sparse_core_info()
_NC = _info.num_cores
_NS = _info.num_subcores
_NW = _NC * _NS
_BPW = TOTAL // _NW

_NBUF = 4
_LEAD = 2
_CHUNK = 400
_NCHUNK = _BPW // _CHUNK
_NGROUP = _NCHUNK // _NBUF

_mesh = plsc.VectorSubcoreMesh(core_axis_name="c", subcore_axis_name="s")


@functools.partial(
    pl.kernel,
    mesh=_mesh,
    out_type=jax.ShapeDtypeStruct((TOTAL, EMBED_DIM), jnp.float32),
    scratch_types=[
        pltpu.VMEM((_BPW,), jnp.int32),
        pltpu.VMEM((_NBUF, _CHUNK, EMBED_DIM), jnp.float32),
        pltpu.SemaphoreType.DMA,
        pltpu.SemaphoreType.DMA,
        pltpu.SemaphoreType.DMA,
        pltpu.SemaphoreType.DMA,
        pltpu.SemaphoreType.DMA,
        pltpu.SemaphoreType.DMA,
        pltpu.SemaphoreType.DMA,
        pltpu.SemaphoreType.DMA,
    ],
    compiler_params=pltpu.CompilerParams(use_tc_tiling_on_sc=False),
)
def _emb_lookup(idx_hbm, table_hbm, out_hbm, idx_all, rows,
                g0, g1, g2, g3, w0, w1, w2, w3):
    gsems = (g0, g1, g2, g3)
    wsems = (w0, w1, w2, w3)
    wid = lax.axis_index("s") * _NC + lax.axis_index("c")
    base0 = wid * _BPW

    pltpu.sync_copy(idx_hbm.at[pl.ds(base0, _BPW)], idx_all)

    def start_gather(slot, chunk):
        pltpu.async_copy(
            table_hbm.at[idx_all.at[pl.ds(chunk * _CHUNK, _CHUNK)]],
            rows.at[slot], gsems[slot])

    def wait_gather(slot):
        pltpu.make_async_copy(
            table_hbm.at[idx_all.at[pl.ds(0, _CHUNK)]],
            rows.at[slot], gsems[slot]).wait()

    def start_wb(slot, chunk):
        pltpu.async_copy(
            rows.at[slot],
            out_hbm.at[pl.ds(base0 + chunk * _CHUNK, _CHUNK)], wsems[slot])

    def wait_wb(slot):
        pltpu.make_async_copy(
            rows.at[slot],
            out_hbm.at[pl.ds(base0, _CHUNK)], wsems[slot]).wait()

    for b in range(_LEAD):
        start_gather(b, b)

    def group(g, carry):
        for b in range(_NBUF):
            i = g * _NBUF + b
            wait_gather(b)
            start_wb(b, i)
            j = i + _LEAD
            sj = (b + _LEAD) % _NBUF

            @pl.when(j < _NCHUNK)
            def _():
                @pl.when(j >= _NBUF)
                def _():
                    wait_wb(sj)
                start_gather(sj, j)
        return carry

    lax.fori_loop(0, _NGROUP, group, 0)

    for b in range(_NBUF):
        wait_wb(b)


def kernel(questions_tensor, table):
    flat_idx = questions_tensor.reshape(TOTAL)
    out = _emb_lookup(flat_idx, table)
    return out.reshape(BATCH, SEQ, EMBED_DIM)


# speedup vs baseline: 1.9686x; 1.0510x over previous
import functools

import jax
import jax.numpy as jnp
from jax import lax
from jax.experimental import pallas as pl
from jax.experimental.pallas import tpu as pltpu
from jax.experimental.pallas import tpu_sc as plsc

BATCH = 16384
SEQ = 50
EMBED_DIM = 64
VOCAB_ROWS = 1000000

_info = plsc.get_sparse_core_info()
_NC = _info.num_cores
_NS = _info.num_subcores
_NW = _NC * _NS
_RPW = BATCH // _NW

_NBUF = 4
_LEAD = 2
_CROWS = 8
_NCHUNK = _RPW // _CROWS
_NGROUP = _NCHUNK // _NBUF

_mesh = plsc.VectorSubcoreMesh(core_axis_name="c", subcore_axis_name="s")


@functools.partial(
    pl.kernel,
    mesh=_mesh,
    out_type=jax.ShapeDtypeStruct((BATCH, SEQ, EMBED_DIM), jnp.float32),
    scratch_types=[
        pltpu.VMEM((_RPW, SEQ), jnp.int32),
        pltpu.VMEM((_NBUF, _CROWS, SEQ, EMBED_DIM), jnp.float32),
        pltpu.SemaphoreType.DMA,
        pltpu.SemaphoreType.DMA,
        pltpu.SemaphoreType.DMA,
        pltpu.SemaphoreType.DMA,
        pltpu.SemaphoreType.DMA,
        pltpu.SemaphoreType.DMA,
        pltpu.SemaphoreType.DMA,
        pltpu.SemaphoreType.DMA,
    ],
    compiler_params=pltpu.CompilerParams(use_tc_tiling_on_sc=False),
)
def _emb_lookup(idx_hbm, table_hbm, out_hbm, idx_all, rows,
                g0, g1, g2, g3, w0, w1, w2, w3):
    gsems = (g0, g1, g2, g3)
    wsems = (w0, w1, w2, w3)
    wid = lax.axis_index("s") * _NC + lax.axis_index("c")
    row0 = wid * _RPW

    pltpu.sync_copy(idx_hbm.at[pl.ds(row0, _RPW), :], idx_all)

    def start_gather(slot, chunk):
        for r in range(_CROWS):
            pltpu.async_copy(
                table_hbm.at[idx_all.at[chunk * _CROWS + r, :]],
                rows.at[slot].at[r], gsems[slot])

    def wait_gather(slot):
        for r in range(_CROWS):
            pltpu.make_async_copy(
                table_hbm.at[idx_all.at[0, :]],
                rows.at[slot].at[r], gsems[slot]).wait()

    def start_wb(slot, chunk):
        pltpu.async_copy(
            rows.at[slot],
            out_hbm.at[pl.ds(row0 + chunk * _CROWS, _CROWS), :, :],
            wsems[slot])

    def wait_wb(slot):
        pltpu.make_async_copy(
            rows.at[slot],
            out_hbm.at[pl.ds(row0, _CROWS), :, :], wsems[slot]).wait()

    for b in range(_LEAD):
        start_gather(b, b)

    def group(g, carry):
        for b in range(_NBUF):
            i = g * _NBUF + b
            wait_gather(b)
            start_wb(b, i)
            j = i + _LEAD
            sj = (b + _LEAD) % _NBUF

            @pl.when(j < _NCHUNK)
            def _():
                @pl.when(j >= _NBUF)
                def _():
                    wait_wb(sj)
                start_gather(sj, j)
        return carry

    lax.fori_loop(0, _NGROUP, group, 0)

    for b in range(_NBUF):
        wait_wb(b)


def kernel(questions_tensor, table):
    q2 = questions_tensor * 2
    tpad = jnp.pad(table, ((0, 0), (0, 64)))
    t2 = tpad.reshape(2 * VOCAB_ROWS, EMBED_DIM)
    return _emb_lookup(q2, t2)
